# Initial kernel scaffold; baseline (speedup 1.0000x reference)
import jax
import jax.numpy as jnp
from jax.experimental import pallas as pl


def kernel(features, xyz, xyz_upsampled):
    raise NotImplementedError("write your pallas kernel here")


# speedup vs baseline: 18.8313x; 18.8313x over previous
import functools

import jax
import jax.numpy as jnp
from jax import lax
from jax.experimental import pallas as pl
from jax.experimental.pallas import tpu as pltpu
from jax.experimental.pallas import tpu_sc as plsc

B, F, N1, N2, K = 4, 256, 2048, 8192, 8
TQ = 256
EPS = 1e-7

NC, NS, L = 2, 16, 16
NW = NC * NS
QTOT = B * N2
QW = QTOT // NW
CQ = 16
NCHUNK = QW // CQ


def _knn_body(xyzt_ref, q_ref, idx_ref, w_ref):
    b = pl.program_id(0)
    q = q_ref[0]
    d2 = None
    for d in range(3):
        qd = q[:, d:d + 1]
        sd = xyzt_ref[0, d:d + 1, :]
        t = qd - sd
        d2 = t * t if d2 is None else d2 + t * t
    iota = lax.broadcasted_iota(jnp.int32, (TQ, N1), 1)
    ms, ams = [], []
    for _ in range(K):
        m = jnp.min(d2, axis=1, keepdims=True)
        am = jnp.min(jnp.where(d2 == m, iota, N1), axis=1,
                     keepdims=True)
        ms.append(m)
        ams.append(am)
        d2 = jnp.where(iota == am, jnp.float32(jnp.inf), d2)
    m8 = jnp.concatenate(ms, axis=1)
    am8 = jnp.concatenate(ams, axis=1)
    dist = jnp.sqrt(jnp.maximum(m8, 0.0))
    wt = (1.0 + EPS) / (dist + EPS)
    wt = wt / jnp.sum(wt, axis=1, keepdims=True)
    idx_ref[0] = am8 + b * N1
    w_ref[0] = jnp.concatenate([wt, jnp.zeros((TQ, 16 - K), jnp.float32)],
                               axis=1)


def _knn(xyz_t, xyz_up):
    return pl.pallas_call(
        _knn_body,
        grid=(B, N2 // TQ),
        in_specs=[
            pl.BlockSpec((1, 3, N1), lambda b, j: (b, 0, 0)),
            pl.BlockSpec((1, TQ, 3), lambda b, j: (b, j, 0)),
        ],
        out_specs=[
            pl.BlockSpec((1, TQ, K), lambda b, j: (b, j, 0)),
            pl.BlockSpec((1, TQ, 16), lambda b, j: (b, j, 0)),
        ],
        out_shape=[
            jax.ShapeDtypeStruct((B, N2, K), jnp.int32),
            jax.ShapeDtypeStruct((B, N2, 16), jnp.float32),
        ],
    )(xyz_t, xyz_up)


@functools.partial(
    pl.kernel,
    mesh=plsc.VectorSubcoreMesh(core_axis_name="c", subcore_axis_name="s"),
    out_type=jax.ShapeDtypeStruct((QTOT, F), jnp.float32),
    scratch_types=[
        pltpu.VMEM((CQ * K,), jnp.int32),
        pltpu.VMEM((CQ * 16,), jnp.float32),
        pltpu.VMEM((CQ * K, F), jnp.float32),
        pltpu.VMEM((CQ, F), jnp.float32),
        pltpu.SemaphoreType.DMA,
    ],
)
def _combine(feat_hbm, idx_hbm, w_hbm, out_hbm, idx_v, w_v, rows_v, out_v,
             sem):
    wid = lax.axis_index("s") * NC + lax.axis_index("c")

    def chunk_body(c, carry):
        base = wid * QW + c * CQ
        pltpu.sync_copy(idx_hbm.at[pl.ds(base * K, CQ * K)], idx_v)
        pltpu.sync_copy(w_hbm.at[pl.ds(base * 16, CQ * 16)], w_v)
        pltpu.async_copy(feat_hbm.at[idx_v], rows_v, sem).wait()

        def qbody(q, carry2):
            qi = q * K
            wvec = w_v[pl.ds(q * 16, 16)]
            wks = [wvec[k] for k in range(K)]
            for j in range(F // L):
                sl = pl.ds(j * L, L)
                acc = wks[0] * rows_v[qi, sl]
                for k in range(1, K):
                    acc = acc + wks[k] * rows_v[qi + k, sl]
                out_v[q, sl] = acc
            return carry2

        lax.fori_loop(0, CQ, qbody, 0)
        pltpu.sync_copy(out_v, out_hbm.at[pl.ds(base, CQ)])
        return carry

    lax.fori_loop(0, NCHUNK, chunk_body, 0)


def kernel(features, xyz, xyz_upsampled):
    xyz_t = jnp.transpose(xyz, (0, 2, 1))
    feat2d = jnp.transpose(features[..., 0], (0, 2, 1))
    feat2d = feat2d.reshape(B * N1, F)

    idx, w = _knn(xyz_t, xyz_upsampled)
    out = _combine(feat2d, idx.reshape(QTOT * K), w.reshape(QTOT * 16))
    out = out.reshape(B, N2, F)
    return jnp.transpose(out, (0, 2, 1))[..., None]


# speedup vs baseline: 20.8956x; 1.1096x over previous
import functools

import jax
import jax.numpy as jnp
from jax import lax
from jax.experimental import pallas as pl
from jax.experimental.pallas import tpu as pltpu
from jax.experimental.pallas import tpu_sc as plsc

B, F, N1, N2, K = 4, 256, 2048, 8192, 8
TQ = 256
EPS = 1e-7

NC, NS, L = 2, 16, 16
NW = NC * NS
QTOT = B * N2
QW = QTOT // NW
CQ = 16
NCHUNK = QW // CQ


def _knn_body(xyzt_ref, q_ref, idx_ref, w_ref):
    b = pl.program_id(0)
    q = q_ref[0]
    d2 = None
    for d in range(3):
        qd = q[:, d:d + 1]
        sd = xyzt_ref[0, d:d + 1, :]
        t = qd - sd
        d2 = t * t if d2 is None else d2 + t * t
    iota = lax.broadcasted_iota(jnp.int32, (TQ, N1), 1)
    ms, ams = [], []
    for _ in range(K):
        m = jnp.min(d2, axis=1, keepdims=True)
        am = jnp.min(jnp.where(d2 == m, iota, N1), axis=1,
                     keepdims=True)
        ms.append(m)
        ams.append(am)
        d2 = jnp.where(iota == am, jnp.float32(jnp.inf), d2)
    m8 = jnp.concatenate(ms, axis=1)
    am8 = jnp.concatenate(ams, axis=1)
    dist = jnp.sqrt(jnp.maximum(m8, 0.0))
    wt = (1.0 + EPS) / (dist + EPS)
    wt = wt / jnp.sum(wt, axis=1, keepdims=True)
    idx_ref[0] = am8 + b * N1
    w_ref[0] = jnp.concatenate([wt, jnp.zeros((TQ, 16 - K), jnp.float32)],
                               axis=1)


def _knn(xyz_t, xyz_up):
    return pl.pallas_call(
        _knn_body,
        grid=(B, N2 // TQ),
        in_specs=[
            pl.BlockSpec((1, 3, N1), lambda b, j: (b, 0, 0)),
            pl.BlockSpec((1, TQ, 3), lambda b, j: (b, j, 0)),
        ],
        out_specs=[
            pl.BlockSpec((1, TQ, K), lambda b, j: (b, j, 0)),
            pl.BlockSpec((1, TQ, 16), lambda b, j: (b, j, 0)),
        ],
        out_shape=[
            jax.ShapeDtypeStruct((B, N2, K), jnp.int32),
            jax.ShapeDtypeStruct((B, N2, 16), jnp.float32),
        ],
    )(xyz_t, xyz_up)


@functools.partial(
    pl.kernel,
    mesh=plsc.VectorSubcoreMesh(core_axis_name="c", subcore_axis_name="s"),
    out_type=jax.ShapeDtypeStruct((QTOT, F), jnp.float32),
    scratch_types=[
        pltpu.VMEM((2, CQ * K), jnp.int32),
        pltpu.VMEM((2, CQ * 16), jnp.float32),
        pltpu.VMEM((2, CQ * K, F), jnp.float32),
        pltpu.VMEM((CQ, F), jnp.float32),
        pltpu.SemaphoreType.DMA,
        pltpu.SemaphoreType.DMA,
    ],
)
def _combine(feat_hbm, idx_hbm, w_hbm, out_hbm, idx_v, w_v, rows_v, out_v,
             sem0, sem1):
    wid = lax.axis_index("s") * NC + lax.axis_index("c")
    qbase = wid * QW
    sems = (sem0, sem1)

    def stage(c, slot):
        base = qbase + c * CQ
        pltpu.sync_copy(idx_hbm.at[pl.ds(base * K, CQ * K)], idx_v.at[slot])
        pltpu.sync_copy(w_hbm.at[pl.ds(base * 16, CQ * 16)], w_v.at[slot])
        pltpu.async_copy(feat_hbm.at[idx_v.at[slot]], rows_v.at[slot],
                         sems[slot])

    stage(0, 0)
    stage(1, 1)

    def outer(c0, carry):
        for slot in range(2):
            c = c0 * 2 + slot
            pltpu.make_async_copy(feat_hbm.at[idx_v.at[slot]],
                                  rows_v.at[slot], sems[slot]).wait()

            def qbody(q, carry2, _slot=slot):
                qi = q * K
                wvec = w_v[_slot, pl.ds(q * 16, 16)]
                wks = [wvec[k] for k in range(K)]
                for j in range(F // L):
                    sl = pl.ds(j * L, L)
                    acc = wks[0] * rows_v[_slot, qi, sl]
                    for k in range(1, K):
                        acc = acc + wks[k] * rows_v[_slot, qi + k, sl]
                    out_v[q, sl] = acc
                return carry2

            lax.fori_loop(0, CQ, qbody, 0)
            pltpu.sync_copy(out_v, out_hbm.at[pl.ds(qbase + c * CQ, CQ)])

            @pl.when(c + 2 < NCHUNK)
            def _():
                stage(c + 2, slot)
        return carry

    lax.fori_loop(0, NCHUNK // 2, outer, 0)


def kernel(features, xyz, xyz_upsampled):
    xyz_t = jnp.transpose(xyz, (0, 2, 1))
    feat2d = jnp.transpose(features[..., 0], (0, 2, 1))
    feat2d = feat2d.reshape(B * N1, F)

    idx, w = _knn(xyz_t, xyz_upsampled)
    out = _combine(feat2d, idx.reshape(QTOT * K), w.reshape(QTOT * 16))
    out = out.reshape(B, N2, F)
    return jnp.transpose(out, (0, 2, 1))[..., None]


# speedup vs baseline: 25.8333x; 1.2363x over previous
import functools

import jax
import jax.numpy as jnp
from jax import lax
from jax.experimental import pallas as pl
from jax.experimental.pallas import tpu as pltpu
from jax.experimental.pallas import tpu_sc as plsc

B, F, N1, N2, K = 4, 256, 2048, 8192, 8
TQ = 512
EPS = 1e-7

NC, NS, L = 2, 16, 16
NW = NC * NS
QTOT = B * N2
QW = QTOT // NW
CQ = 16
NCHUNK = QW // CQ


def _knn_body(xyzt_ref, q_ref, idx_ref, w_ref):
    b = pl.program_id(0)
    q = q_ref[0]
    st = xyzt_ref[0]
    qs2 = lax.dot_general(q * (-2.0), st, (((1,), (0,)), ((), ())),
                          preferred_element_type=jnp.float32)
    q2 = jnp.sum(q * q, axis=1, keepdims=True)
    s2 = jnp.sum(st * st, axis=0, keepdims=True)
    d2 = (qs2 + q2) + s2
    iota = lax.broadcasted_iota(jnp.int32, (TQ, N1), 1).astype(jnp.float32)
    ms, ams = [], []
    for _ in range(K):
        m = jnp.min(d2, axis=1, keepdims=True)
        am = jnp.min(jnp.where(d2 == m, iota, jnp.float32(N1)), axis=1,
                     keepdims=True)
        ms.append(m)
        ams.append(am)
        d2 = jnp.where(iota == am, jnp.float32(jnp.inf), d2)
    m8 = jnp.concatenate(ms, axis=1)
    am8 = jnp.minimum(jnp.concatenate(ams, axis=1),
                      jnp.float32(N1 - 1)).astype(jnp.int32)
    dist = jnp.sqrt(jnp.maximum(m8, 0.0))
    wt = (1.0 + EPS) / (dist + EPS)
    wt = wt / jnp.sum(wt, axis=1, keepdims=True)
    idx_ref[0] = am8 + b * N1
    w_ref[0] = jnp.concatenate([wt, jnp.zeros((TQ, 16 - K), jnp.float32)],
                               axis=1)


def _knn(xyz_t, xyz_up):
    return pl.pallas_call(
        _knn_body,
        grid=(B, N2 // TQ),
        in_specs=[
            pl.BlockSpec((1, 3, N1), lambda b, j: (b, 0, 0)),
            pl.BlockSpec((1, TQ, 3), lambda b, j: (b, j, 0)),
        ],
        out_specs=[
            pl.BlockSpec((1, TQ, K), lambda b, j: (b, j, 0)),
            pl.BlockSpec((1, TQ, 16), lambda b, j: (b, j, 0)),
        ],
        out_shape=[
            jax.ShapeDtypeStruct((B, N2, K), jnp.int32),
            jax.ShapeDtypeStruct((B, N2, 16), jnp.float32),
        ],
    )(xyz_t, xyz_up)


@functools.partial(
    pl.kernel,
    mesh=plsc.VectorSubcoreMesh(core_axis_name="c", subcore_axis_name="s"),
    out_type=jax.ShapeDtypeStruct((QTOT, F), jnp.float32),
    scratch_types=[
        pltpu.VMEM((2, CQ * K), jnp.int32),
        pltpu.VMEM((2, CQ * 16), jnp.float32),
        pltpu.VMEM((2, CQ * K, F), jnp.float32),
        pltpu.VMEM((CQ, F), jnp.float32),
        pltpu.SemaphoreType.DMA,
        pltpu.SemaphoreType.DMA,
    ],
)
def _combine(feat_hbm, idx_hbm, w_hbm, out_hbm, idx_v, w_v, rows_v, out_v,
             sem0, sem1):
    wid = lax.axis_index("s") * NC + lax.axis_index("c")
    qbase = wid * QW
    sems = (sem0, sem1)

    def stage(c, slot):
        base = qbase + c * CQ
        pltpu.sync_copy(idx_hbm.at[pl.ds(base * K, CQ * K)], idx_v.at[slot])
        pltpu.sync_copy(w_hbm.at[pl.ds(base * 16, CQ * 16)], w_v.at[slot])
        pltpu.async_copy(feat_hbm.at[idx_v.at[slot]], rows_v.at[slot],
                         sems[slot])

    stage(0, 0)
    stage(1, 1)

    def outer(c0, carry):
        for slot in range(2):
            c = c0 * 2 + slot
            pltpu.make_async_copy(feat_hbm.at[idx_v.at[slot]],
                                  rows_v.at[slot], sems[slot]).wait()

            def qbody(q, carry2, _slot=slot):
                qi = q * K
                wvec = w_v[_slot, pl.ds(q * 16, 16)]
                wks = [wvec[k] for k in range(K)]
                for j in range(F // L):
                    sl = pl.ds(j * L, L)
                    acc = wks[0] * rows_v[_slot, qi, sl]
                    for k in range(1, K):
                        acc = acc + wks[k] * rows_v[_slot, qi + k, sl]
                    out_v[q, sl] = acc
                return carry2

            lax.fori_loop(0, CQ, qbody, 0)
            pltpu.sync_copy(out_v, out_hbm.at[pl.ds(qbase + c * CQ, CQ)])

            @pl.when(c + 2 < NCHUNK)
            def _():
                stage(c + 2, slot)
        return carry

    lax.fori_loop(0, NCHUNK // 2, outer, 0)


def kernel(features, xyz, xyz_upsampled):
    xyz_t = jnp.transpose(xyz, (0, 2, 1))
    feat2d = jnp.transpose(features[..., 0], (0, 2, 1))
    feat2d = feat2d.reshape(B * N1, F)

    idx, w = _knn(xyz_t, xyz_upsampled)
    out = _combine(feat2d, idx.reshape(QTOT * K), w.reshape(QTOT * 16))
    out = out.reshape(B, N2, F)
    return jnp.transpose(out, (0, 2, 1))[..., None]
